# Initial kernel scaffold; baseline (speedup 1.0000x reference)
import jax
import jax.numpy as jnp
from jax.experimental import pallas as pl


def kernel(sparse_codes, edge_index, pattern_weights):
    raise NotImplementedError("write your pallas kernel here")


# speedup vs baseline: 2.9180x; 2.9180x over previous
import dataclasses
import functools

import jax
import jax.numpy as jnp
from jax import lax
from jax.experimental import pallas as pl
from jax.experimental.pallas import tpu as pltpu
from jax.experimental.pallas import tpu_sc as plsc

N_NODES = 10000
N_EDGES = 320000
NUM_ATOMS = 128

NC = 2
NS = 16
NW = NC * NS
LANES = 16
EPW = N_EDGES // NW
G = 80
NCHUNK = EPW // G
GROUPS = G // LANES


def _prescale_body(codes_ref, w_ref, out_ref):
    out_ref[...] = codes_ref[...] * w_ref[...]


def _prescale(codes, w):
    return pl.pallas_call(
        _prescale_body,
        out_shape=jax.ShapeDtypeStruct((N_NODES, NUM_ATOMS), jnp.float32),
    )(codes, w.reshape(1, NUM_ATOMS))


def _edge_score_sc(table_scaled, table_raw, src_idx, dst_idx):
    mesh = plsc.VectorSubcoreMesh(core_axis_name="c", subcore_axis_name="s")
    cp = pltpu.CompilerParams()
    if "needs_layout_passes" in pltpu.CompilerParams.__dataclass_fields__:
        cp = dataclasses.replace(cp, needs_layout_passes=False)

    @functools.partial(
        pl.kernel,
        mesh=mesh,
        compiler_params=cp,
        out_type=jax.ShapeDtypeStruct((N_EDGES,), jnp.float32),
        scratch_types=[
            pltpu.VMEM((G,), jnp.int32),
            pltpu.VMEM((G,), jnp.int32),
            pltpu.VMEM((G, NUM_ATOMS), jnp.float32),
            pltpu.VMEM((G, NUM_ATOMS), jnp.float32),
            pltpu.VMEM((G * LANES,), jnp.float32),
            pltpu.VMEM((G,), jnp.float32),
            pltpu.SemaphoreType.DMA,
        ],
    )
    def k(ts_hbm, tr_hbm, sidx_hbm, didx_hbm, out_hbm,
          sidx_v, didx_v, srows_v, drows_v, part_v, out_v, sem):
        wid = lax.axis_index("s") * NC + lax.axis_index("c")
        tile_base = wid * EPW

        @pl.loop(0, NCHUNK)
        def _(c):
            base = tile_base + c * G
            pltpu.sync_copy(sidx_hbm.at[pl.ds(base, G)], sidx_v)
            pltpu.sync_copy(didx_hbm.at[pl.ds(base, G)], didx_v)
            pltpu.async_copy(ts_hbm.at[sidx_v], srows_v, sem).wait()
            pltpu.async_copy(tr_hbm.at[didx_v], drows_v, sem).wait()

            @pl.loop(0, G)
            def _(e):
                acc = srows_v[e, pl.ds(0, LANES)] * drows_v[e, pl.ds(0, LANES)]
                for j in range(1, NUM_ATOMS // LANES):
                    acc = jnp.maximum(
                        acc,
                        srows_v[e, pl.ds(j * LANES, LANES)]
                        * drows_v[e, pl.ds(j * LANES, LANES)],
                    )
                part_v[pl.ds(e * LANES, LANES)] = acc

            iota = lax.iota(jnp.int32, LANES)

            @pl.loop(0, GROUPS)
            def _(t):
                col = t * (LANES * LANES) + iota * LANES
                m = plsc.load_gather(part_v, [col])
                for l in range(1, LANES):
                    m = jnp.maximum(m, plsc.load_gather(part_v, [col + l]))
                out_v[pl.ds(t * LANES, LANES)] = 1.0 / (1.0 + jnp.exp(-m))

            pltpu.sync_copy(out_v, out_hbm.at[pl.ds(base, G)])

    return k(table_scaled, table_raw, src_idx, dst_idx)


def kernel(sparse_codes, edge_index, pattern_weights):
    scaled = _prescale(sparse_codes, pattern_weights)
    src_idx = edge_index[0].astype(jnp.int32)
    dst_idx = edge_index[1].astype(jnp.int32)
    return _edge_score_sc(scaled, sparse_codes, src_idx, dst_idx)


# speedup vs baseline: 5.3586x; 1.8364x over previous
import dataclasses
import functools

import jax
import jax.numpy as jnp
from jax import lax
from jax.experimental import pallas as pl
from jax.experimental.pallas import tpu as pltpu
from jax.experimental.pallas import tpu_sc as plsc

N_NODES = 10000
N_EDGES = 320000
NUM_ATOMS = 128

NC = 2
NS = 16
NW = NC * NS
LANES = 16
EPW = N_EDGES // NW
G = 80
NCHUNK = EPW // G
GROUPS = G // LANES


def _prescale_body(codes_ref, w_ref, out_ref):
    out_ref[...] = codes_ref[...] * w_ref[...]


def _prescale(codes, w):
    return pl.pallas_call(
        _prescale_body,
        out_shape=jax.ShapeDtypeStruct((N_NODES, NUM_ATOMS), jnp.float32),
    )(codes, w.reshape(1, NUM_ATOMS))


def _edge_score_sc(table_scaled, table_raw, src_idx, dst_idx):
    mesh = plsc.VectorSubcoreMesh(core_axis_name="c", subcore_axis_name="s")
    cp = pltpu.CompilerParams()
    if "needs_layout_passes" in pltpu.CompilerParams.__dataclass_fields__:
        cp = dataclasses.replace(cp, needs_layout_passes=False)

    @functools.partial(
        pl.kernel,
        mesh=mesh,
        compiler_params=cp,
        out_type=jax.ShapeDtypeStruct((N_EDGES,), jnp.float32),
        scratch_types=[
            pltpu.VMEM((G,), jnp.int32),
            pltpu.VMEM((G,), jnp.int32),
            pltpu.VMEM((G,), jnp.int32),
            pltpu.VMEM((G,), jnp.int32),
            pltpu.VMEM((G, NUM_ATOMS), jnp.float32),
            pltpu.VMEM((G, NUM_ATOMS), jnp.float32),
            pltpu.VMEM((G, NUM_ATOMS), jnp.float32),
            pltpu.VMEM((G, NUM_ATOMS), jnp.float32),
            pltpu.VMEM((G * LANES,), jnp.float32),
            pltpu.VMEM((G,), jnp.float32),
            pltpu.VMEM((G,), jnp.float32),
            pltpu.SemaphoreType.DMA,
            pltpu.SemaphoreType.DMA,
            pltpu.SemaphoreType.DMA,
            pltpu.SemaphoreType.DMA,
        ],
    )
    def k(ts_hbm, tr_hbm, sidx_hbm, didx_hbm, out_hbm,
          sidxA, didxA, sidxB, didxB, srA, drA, srB, drB, part_v,
          outA, outB, semgA, semgB, semoA, semoB):
        wid = lax.axis_index("s") * NC + lax.axis_index("c")
        tile_base = wid * EPW
        bufs = {
            0: (sidxA, didxA, srA, drA, outA, semgA, semoA),
            1: (sidxB, didxB, srB, drB, outB, semgB, semoB),
        }

        def issue(chunk, b):
            sidx, didx, sr, dr, _, semg, _ = bufs[b]
            base = tile_base + chunk * G
            pltpu.sync_copy(sidx_hbm.at[pl.ds(base, G)], sidx)
            pltpu.sync_copy(didx_hbm.at[pl.ds(base, G)], didx)
            pltpu.async_copy(ts_hbm.at[sidx], sr, semg)
            pltpu.async_copy(tr_hbm.at[didx], dr, semg)

        def compute(chunk, b):
            sidx, didx, sr, dr, outb, semg, semo = bufs[b]
            pltpu.make_async_copy(ts_hbm.at[sidx], sr, semg).wait()
            pltpu.make_async_copy(tr_hbm.at[didx], dr, semg).wait()

            @pl.when(chunk >= 2)
            def _():
                pltpu.make_async_copy(
                    outb, out_hbm.at[pl.ds(tile_base, G)], semo).wait()

            @pl.loop(0, G)
            def _(e):
                acc = sr[e, pl.ds(0, LANES)] * dr[e, pl.ds(0, LANES)]
                for j in range(1, NUM_ATOMS // LANES):
                    acc = jnp.maximum(
                        acc,
                        sr[e, pl.ds(j * LANES, LANES)]
                        * dr[e, pl.ds(j * LANES, LANES)],
                    )
                part_v[pl.ds(e * LANES, LANES)] = acc

            iota = lax.iota(jnp.int32, LANES)

            @pl.loop(0, GROUPS)
            def _(t):
                col = t * (LANES * LANES) + iota * LANES
                m = plsc.load_gather(part_v, [col])
                for l in range(1, LANES):
                    m = jnp.maximum(m, plsc.load_gather(part_v, [col + l]))
                outb[pl.ds(t * LANES, LANES)] = 1.0 / (1.0 + jnp.exp(-m))

            pltpu.async_copy(
                outb, out_hbm.at[pl.ds(tile_base + chunk * G, G)], semo)

        issue(0, 0)

        @pl.loop(0, NCHUNK, step=2)
        def _(c):
            @pl.when(c + 1 < NCHUNK)
            def _():
                issue(c + 1, 1)

            compute(c, 0)

            @pl.when(c + 2 < NCHUNK)
            def _():
                issue(c + 2, 0)

            @pl.when(c + 1 < NCHUNK)
            def _():
                compute(c + 1, 1)

        pltpu.make_async_copy(outA, out_hbm.at[pl.ds(tile_base, G)], semoA).wait()
        pltpu.make_async_copy(outB, out_hbm.at[pl.ds(tile_base, G)], semoB).wait()

    return k(table_scaled, table_raw, src_idx, dst_idx)


def kernel(sparse_codes, edge_index, pattern_weights):
    scaled = _prescale(sparse_codes, pattern_weights)
    src_idx = edge_index[0].astype(jnp.int32)
    dst_idx = edge_index[1].astype(jnp.int32)
    return _edge_score_sc(scaled, sparse_codes, src_idx, dst_idx)
